# Initial kernel scaffold; baseline (speedup 1.0000x reference)
import jax
import jax.numpy as jnp
from jax.experimental import pallas as pl


def kernel(rpn_probs, rpn_bbox, anchors):
    raise NotImplementedError("write your pallas kernel here")


# speedup vs baseline: 13.6828x; 13.6828x over previous
import functools

import jax
import jax.numpy as jnp
from jax.experimental import pallas as pl

_PRE_NMS_LIMIT = 6000
_PROPOSAL_COUNT = 1000
_NMS_THRESHOLD = 0.7
_STD = (0.1, 0.1, 0.2, 0.2)
_LANES = 128


def _proposal_kernel(s_ref, b0_ref, b1_ref, b2_ref, b3_ref,
                     a0_ref, a1_ref, a2_ref, a3_ref,
                     oy1_ref, ox1_ref, oy2_ref, ox2_ref, osc_ref):
    B, R, L = s_ref.shape
    s = s_ref[...]

    a0 = a0_ref[...]
    a1 = a1_ref[...]
    a2 = a2_ref[...]
    a3 = a3_ref[...]
    h = a2 - a0
    w = a3 - a1
    cy = a0 + 0.5 * h
    cx = a1 + 0.5 * w
    cy = cy + (b0_ref[...] * _STD[0]) * h
    cx = cx + (b1_ref[...] * _STD[1]) * w
    hh = h * jnp.exp(b2_ref[...] * _STD[2])
    ww = w * jnp.exp(b3_ref[...] * _STD[3])
    y1 = cy - 0.5 * hh
    x1 = cx - 0.5 * ww
    y2 = y1 + hh
    x2 = x1 + ww
    y1 = jnp.clip(y1, 0.0, 1.0)
    x1 = jnp.clip(x1, 0.0, 1.0)
    y2 = jnp.clip(y2, 0.0, 1.0)
    x2 = jnp.clip(x2, 0.0, 1.0)
    areas = (y2 - y1) * (x2 - x1)

    pos = (jax.lax.broadcasted_iota(jnp.int32, (B, R, L), 1) * L
           + jax.lax.broadcasted_iota(jnp.int32, (B, R, L), 2))

    def bit_step(j, cur):
        bit = jnp.left_shift(jnp.int32(1), 30 - j)
        trial = jnp.bitwise_or(cur, bit)
        x = jax.lax.bitcast_convert_type(trial, jnp.float32)
        cnt = jnp.sum(jnp.where(s >= x, 1.0, 0.0), axis=(1, 2), keepdims=True)
        return jnp.where(cnt >= jnp.float32(_PRE_NMS_LIMIT), trial, cur)

    vbits = jax.lax.fori_loop(0, 31, bit_step,
                              jnp.zeros((B, 1, 1), jnp.int32))
    v = jax.lax.bitcast_convert_type(vbits, jnp.float32)

    gt = s > v
    eq = s == v
    count_gt = jnp.sum(jnp.where(gt, 1.0, 0.0), axis=(1, 2), keepdims=True)
    need = jnp.float32(_PRE_NMS_LIMIT) - count_gt

    eqf = jnp.where(eq, 1.0, 0.0).reshape(B * R, L)
    u_in = jnp.where(jax.lax.broadcasted_iota(jnp.int32, (L, L), 0)
                     <= jax.lax.broadcasted_iota(jnp.int32, (L, L), 1),
                     1.0, 0.0)
    rowcs = jax.lax.dot_general(eqf, u_in, (((1,), (0,)), ((), ())),
                                preferred_element_type=jnp.float32)
    rowcs = rowcs.reshape(B, R, L)
    rowtot = rowcs[:, :, L - 1]
    u_ex = jnp.where(jax.lax.broadcasted_iota(jnp.int32, (R, R), 0)
                     < jax.lax.broadcasted_iota(jnp.int32, (R, R), 1),
                     1.0, 0.0)
    offs = jax.lax.dot_general(rowtot, u_ex, (((1,), (0,)), ((), ())),
                               preferred_element_type=jnp.float32)
    cs = rowcs + offs[:, :, None]
    valid0 = jnp.where(gt | (eq & (cs <= need)), 1.0, 0.0)

    zero_out = jnp.zeros((_PROPOSAL_COUNT, B), jnp.float32)
    oy1_ref[...] = zero_out
    ox1_ref[...] = zero_out
    oy2_ref[...] = zero_out
    ox2_ref[...] = zero_out
    osc_ref[...] = zero_out

    NEG = jnp.float32(-1e30)
    BIGI = jnp.int32(2 ** 30)

    def nms_step(i, valid):
        ms = jnp.where(valid > 0.5, s, NEG)
        m1 = jnp.max(ms, axis=(1, 2), keepdims=True)
        has = m1 > jnp.float32(-1e29)
        cand = jnp.where(ms == m1, pos, BIGI)
        idx = jnp.min(cand, axis=(1, 2), keepdims=True)
        one = pos == idx
        by1 = jnp.sum(jnp.where(one, y1, 0.0), axis=(1, 2), keepdims=True)
        bx1 = jnp.sum(jnp.where(one, x1, 0.0), axis=(1, 2), keepdims=True)
        by2 = jnp.sum(jnp.where(one, y2, 0.0), axis=(1, 2), keepdims=True)
        bx2 = jnp.sum(jnp.where(one, x2, 0.0), axis=(1, 2), keepdims=True)
        bsc = jnp.sum(jnp.where(one, s, 0.0), axis=(1, 2), keepdims=True)

        yy1 = jnp.maximum(by1, y1)
        xx1 = jnp.maximum(bx1, x1)
        yy2 = jnp.minimum(by2, y2)
        xx2 = jnp.minimum(bx2, x2)
        inter = (jnp.maximum(yy2 - yy1, 0.0) * jnp.maximum(xx2 - xx1, 0.0))
        barea = (by2 - by1) * (bx2 - bx1)
        union = barea + areas - inter
        iou = inter / jnp.maximum(union, 1e-8)
        keep = (iou <= _NMS_THRESHOLD) & (pos != idx)
        new_valid = jnp.where(keep, valid, 0.0)
        valid = jnp.where(has, new_valid, valid)

        hasf = has
        oy1_ref[pl.ds(i, 1), :] = jnp.where(hasf, by1, 0.0).reshape(1, B)
        ox1_ref[pl.ds(i, 1), :] = jnp.where(hasf, bx1, 0.0).reshape(1, B)
        oy2_ref[pl.ds(i, 1), :] = jnp.where(hasf, by2, 0.0).reshape(1, B)
        ox2_ref[pl.ds(i, 1), :] = jnp.where(hasf, bx2, 0.0).reshape(1, B)
        osc_ref[pl.ds(i, 1), :] = jnp.where(hasf, bsc, 0.0).reshape(1, B)
        return valid

    jax.lax.fori_loop(0, _PROPOSAL_COUNT, nms_step, valid0)


@functools.partial(jax.jit)
def kernel(rpn_probs, rpn_bbox, anchors):
    B, N, _ = rpn_probs.shape
    R = (N + _LANES - 1) // _LANES
    R = ((R + 7) // 8) * 8
    pad = R * _LANES - N

    def prep(x, fill):
        return jnp.pad(x, ((0, 0), (0, pad)),
                       constant_values=fill).reshape(B, R, _LANES)

    s = prep(rpn_probs[:, :, 1], -1.0)
    bb = [prep(rpn_bbox[:, :, k], 0.0) for k in range(4)]
    aa = [prep(anchors[:, :, k], 0.0) for k in range(4)]

    out_sds = [jax.ShapeDtypeStruct((_PROPOSAL_COUNT, B), jnp.float32)] * 5
    oy1, ox1, oy2, ox2, osc = pl.pallas_call(
        _proposal_kernel,
        out_shape=out_sds,
    )(s, *bb, *aa)

    proposals = jnp.stack([oy1.T, ox1.T, oy2.T, ox2.T], axis=-1)
    return proposals, osc.T
